# Initial kernel scaffold; baseline (speedup 1.0000x reference)
import jax
import jax.numpy as jnp
from jax.experimental import pallas as pl


def kernel(x_text, x_audio, x_video, x_z, lin_W, lin_b, conv_Wl, conv_Wr, conv_att, conv_bias, proj_W0, proj_b0, proj_W1, proj_b1, ei_ta, ei_at, ei_tv, ei_vt, ei_av, ei_va, ei_tz, ei_zt, ei_az, ei_za, ei_vz, ei_zv, ei_zz, batch_text, batch_audio, batch_video):
    raise NotImplementedError("write your pallas kernel here")


# speedup vs baseline: 22.1302x; 22.1302x over previous
import functools

import jax
import jax.numpy as jnp
import numpy as np
from jax import lax
from jax.experimental import pallas as pl
from jax.experimental.pallas import tpu as pltpu
from jax.experimental.pallas import tpu_sc as plsc

F32 = jnp.float32
I32 = jnp.int32

D = 128
HEADS = 4
CH = 32
G = 512
TEMP = 0.1
NBIG = 10000
NZ = 512
NPAD = 10240
ACC_BIG = 10240
ACC_Z = 640
GARB_BIG = 10000
GARB_Z = 512
KCH = 128
MAXCH = 48

_TID = {"t": 0, "a": 1, "v": 2, "z": 3}
_ETS = [
    ("ta", "t", "a", 90000), ("at", "a", "t", 90000), ("tv", "t", "v", 90000),
    ("vt", "v", "t", 90000), ("av", "a", "v", 90000), ("va", "v", "a", 90000),
    ("tz", "t", "z", 10000), ("zt", "z", "t", 10000), ("az", "a", "z", 10000),
    ("za", "z", "a", 10000), ("vz", "v", "z", 10000), ("zv", "z", "v", 10000),
    ("zz", "z", "z", 4096),
]
_ENAME2IDX = {nm: i for i, (nm, _, _, _) in enumerate(_ETS)}
_SRC_IDS = [_TID[s] for (_, s, _, _) in _ETS]
_DST_IDS = [_TID[d] for (_, _, d, _) in _ETS]

def _pe_consts():
    lane = lax.broadcasted_iota(I32, (1, 128), 1)
    div = jnp.exp(-np.float32(np.log(10000.0) / 128.0)
                  * (lane // 2 * 2).astype(F32))
    even = lane % 2 == 0
    return div, even


def _msel_const():
    row = lax.broadcasted_iota(I32, (16, 128), 0)
    lane = lax.broadcasted_iota(I32, (16, 128), 1)
    return (lane // CH == row).astype(F32)


def _pos_body(srow_ref, scol_ref, o_ref):
    i = pl.program_id(1)
    j = pl.program_id(2)

    @pl.when(j == 0)
    def _():
        o_ref[0] = (lax.broadcasted_iota(I32, (512, 1), 0)
                    + i * 512).astype(F32)

    seg_i = scol_ref[0]
    seg_j = srow_ref[0]
    cmp = (seg_j < seg_i).astype(F32)
    o_ref[0] = o_ref[0] - jnp.sum(cmp, axis=1, keepdims=True)


def _positions(segrow, segcol):
    return pl.pallas_call(
        _pos_body,
        grid=(3, 20, 20),
        in_specs=[
            pl.BlockSpec((1, 1, 512), lambda m, i, j: (m * 20 + j, 0, 0)),
            pl.BlockSpec((1, 512, 1), lambda m, i, j: (m * 20 + i, 0, 0)),
        ],
        out_specs=pl.BlockSpec((1, 512, 1), lambda m, i, j: (m * 20 + i, 0, 0)),
        out_shape=jax.ShapeDtypeStruct((60, 512, 1), F32),
    )(segrow, segcol)


def _lin_body(flag_ref, x_ref, w_ref, b_ref, p_ref, o_ref):
    j = pl.program_id(0)
    h = jnp.dot(x_ref[0], w_ref[0], preferred_element_type=F32) + b_ref[0, 0:1, :]
    pos = p_ref[0]
    div, even = _pe_consts()
    ang = pos * div
    pe = jnp.where(even, jnp.sin(ang), jnp.cos(ang))
    o_ref[0] = h + pe * flag_ref[j].astype(F32)


def _lin_pe(xstack, lw, lb, post, flags):
    return pl.pallas_call(
        _lin_body,
        grid_spec=pltpu.PrefetchScalarGridSpec(
            num_scalar_prefetch=1,
            grid=(4, 20),
            in_specs=[
                pl.BlockSpec((1, 512, 128), lambda j, b, f: (j, b, 0)),
                pl.BlockSpec((1, 128, 128), lambda j, b, f: (j, 0, 0)),
                pl.BlockSpec((1, 8, 128), lambda j, b, f: (j, 0, 0)),
                pl.BlockSpec((1, 512, 1),
                             lambda j, b, f: (jnp.minimum(j * 20 + b, 59), 0, 0)),
            ],
            out_specs=pl.BlockSpec((1, 512, 128), lambda j, b, f: (j, b, 0)),
        ),
        out_shape=jax.ShapeDtypeStruct((4, NPAD, 128), F32),
    )(flags, xstack, lw, lb, post)


def _mm_body(stab_ref, h_ref, w_ref, o_ref):
    o_ref[0] = jnp.dot(h_ref[0], w_ref[0], preferred_element_type=F32)


def _project26(stab, hstack, w26):
    return pl.pallas_call(
        _mm_body,
        grid_spec=pltpu.PrefetchScalarGridSpec(
            num_scalar_prefetch=1,
            grid=(26, 20),
            in_specs=[
                pl.BlockSpec((1, 512, 128), lambda j, b, s: (s[j], b, 0)),
                pl.BlockSpec((1, 128, 128), lambda j, b, s: (j, 0, 0)),
            ],
            out_specs=pl.BlockSpec((1, 512, 128), lambda j, b, s: (j, b, 0)),
        ),
        out_shape=jax.ShapeDtypeStruct((26, NPAD, 128), F32),
    )(stab, hstack, w26)


def _combine(nds, pairs, bias_sum, inv_k):
    npairs = len(pairs)

    def body(*refs):
        b_ref = refs[2 * npairs]
        o_ref = refs[2 * npairs + 1]
        msel = _msel_const()
        acc = None
        for p in range(npairs):
            num = refs[2 * p][...]
            den = refs[2 * p + 1][...]
            inv = 1.0 / (den + 1e-16)
            o = num * jnp.dot(inv, msel, preferred_element_type=F32)
            acc = o if acc is None else acc + o
        o_ref[...] = jnp.maximum((acc + b_ref[0:1, :]) * inv_k, 0.0)

    ins = []
    specs = []
    for num, den in pairs:
        ins += [num, den]
        specs += [pl.BlockSpec((128, 128), lambda b: (b, 0)),
                  pl.BlockSpec((128, 16), lambda b: (b, 0))]
    ins.append(bias_sum)
    specs.append(pl.BlockSpec((8, 128), lambda b: (0, 0)))
    return pl.pallas_call(
        body,
        grid=(nds // 128,),
        in_specs=specs,
        out_specs=pl.BlockSpec((128, 128), lambda b: (b, 0)),
        out_shape=jax.ShapeDtypeStruct((nds, 128), F32),
    )(*ins)


def _loss_body(p0_ref, p1_ref, segs_ref, w0_ref, b0_ref, w1_ref, b1_ref, o_ref):
    gid = lax.broadcasted_iota(I32, (512, 1), 0).astype(F32)
    cnt = jnp.zeros((512, 1), F32)
    for r in range(60):
        seg_r = segs_ref[r]
        cnt = cnt + jnp.sum((seg_r == gid).astype(F32), axis=1, keepdims=True)
    s = (p0_ref[0:512, :] + p1_ref[0:512, :]) / jnp.maximum(cnt, 1.0)
    x = jnp.maximum(jnp.dot(s, w0_ref[...], preferred_element_type=F32)
                    + b0_ref[0:1, :], 0.0)
    x = jnp.maximum(jnp.dot(x, w1_ref[...], preferred_element_type=F32)
                    + b1_ref[0:1, :], 0.0)
    nrm = jnp.sqrt(jnp.sum(x * x, axis=1, keepdims=True))
    p = x / jnp.maximum(nrm, 1e-12)
    s_mat = lax.dot_general(p, p, (((1,), (1,)), ((), ())),
                            preferred_element_type=F32) * (1.0 / TEMP)
    eye = (lax.broadcasted_iota(I32, (512, 512), 0)
           == lax.broadcasted_iota(I32, (512, 512), 1)).astype(F32)
    masked = s_mat - eye * 1e9
    m = jnp.max(s_mat, axis=1, keepdims=True)
    lse = m + jnp.log(jnp.sum(jnp.exp(s_mat - m), axis=1, keepdims=True)
                      + jnp.sum(jnp.exp(masked - m), axis=1, keepdims=True))
    diag = jnp.sum(s_mat * eye, axis=1, keepdims=True)
    la = lse - diag
    loss = 2.0 * jnp.sum(la) / 512.0
    o_ref[...] = jnp.full((8, 128), loss, F32)


def _loss(pool0, pool1, segs, w0, b0, w1, b1):
    return pl.pallas_call(
        _loss_body,
        out_shape=jax.ShapeDtypeStruct((8, 128), F32),
    )(pool0, pool1, segs, w0, b0, w1, b1)


_MESH = plsc.VectorSubcoreMesh(core_axis_name="c", subcore_axis_name="s",
                               num_cores=2, num_subcores=16)


def _zero_rows(ref, nrows, width):
    zv = jnp.zeros((16,), F32)

    def bd(r, _):
        for i in range(width // 16):
            ref[r, pl.ds(16 * i, 16)] = zv
        return 0

    lax.fori_loop(0, nrows, bd, 0)


@functools.lru_cache(maxsize=None)
def _edge_kernel(cfg):
    njobs = sum(len(jobs) for jobs in cfg)
    out_type = []
    for jobs in cfg:
        for (_, nd_acc, _h) in jobs:
            out_type.append(jax.ShapeDtypeStruct((nd_acc, 64), F32))
            out_type.append(jax.ShapeDtypeStruct((nd_acc, 16), F32))

    @functools.partial(
        pl.kernel,
        out_type=out_type,
        mesh=_MESH,
        compiler_params=pltpu.CompilerParams(needs_layout_passes=False,
                                             use_tc_tiling_on_sc=False),
        scratch_types=[
            pltpu.VMEM((MAXCH, KCH), I32),
            pltpu.VMEM((MAXCH, KCH), I32),
            pltpu.VMEM((KCH, 128), F32),
            pltpu.VMEM((KCH, 128), F32),
            pltpu.VMEM((KCH, 64), F32),
            pltpu.VMEM((KCH, 16), F32),
            pltpu.VMEM((4, 16), F32),
            pltpu.VMEM_SHARED((ACC_BIG, 64), F32),
            pltpu.VMEM_SHARED((ACC_BIG, 16), F32),
            pltpu.SemaphoreType.DMA,
            pltpu.SemaphoreType.DMA,
        ],
    )
    def kern(*refs):
        ins = refs[:5 * njobs]
        outs = refs[5 * njobs:5 * njobs + 2 * njobs]
        (sidx, didx, hsb, hdb, msgb, denb, attv,
         accn, accd, sem0, sem1) = refs[5 * njobs + 2 * njobs:]
        c = lax.axis_index("c")
        s = lax.axis_index("s")
        iot = lax.iota(I32, 16)
        ohs = [(iot == h).astype(F32) for h in range(2)]
        hmask = iot < 2
        zv = jnp.zeros((16,), F32)

        flat = 0
        for core_id, jobs in enumerate(cfg):
            job_in = ins[5 * flat:]
            job_out = outs[2 * flat:]
            flat += len(jobs)

            @pl.when(c == core_id)
            def _(jobs=jobs, job_in=job_in, job_out=job_out):
                for ji, (nchunks, nd_acc, hhalf) in enumerate(jobs):
                    hs_h, hd_h, src_h, dst_h, att_h = job_in[5 * ji:5 * ji + 5]
                    num_o, den_o = job_out[2 * ji:2 * ji + 2]
                    base = 64 * hhalf
                    rpt = nd_acc // 16
                    _zero_rows(msgb, KCH, 64)
                    _zero_rows(denb, KCH, 16)
                    nfull, rem = rpt // KCH, rpt % KCH
                    for q in range(nfull):
                        pltpu.sync_copy(
                            msgb, accn.at[pl.ds(s * rpt + q * KCH, KCH)])
                        pltpu.sync_copy(
                            denb, accd.at[pl.ds(s * rpt + q * KCH, KCH)])
                    if rem:
                        pltpu.sync_copy(
                            msgb.at[pl.ds(0, rem)],
                            accn.at[pl.ds(s * rpt + nfull * KCH, rem)])
                        pltpu.sync_copy(
                            denb.at[pl.ds(0, rem)],
                            accd.at[pl.ds(s * rpt + nfull * KCH, rem)])
                    plsc.subcore_barrier()
                    chpad = -(-nchunks // 8) * 8
                    pltpu.sync_copy(src_h.at[pl.ds(s * chpad, chpad)],
                                    sidx.at[pl.ds(0, chpad)])
                    pltpu.sync_copy(dst_h.at[pl.ds(s * chpad, chpad)],
                                    didx.at[pl.ds(0, chpad)])
                    pltpu.sync_copy(att_h, attv)
                    att = [attv[i, :] for i in range(4)]

                    def chunk(j, _):
                        g1 = pltpu.async_copy(hs_h.at[sidx.at[j]], hsb, sem0)
                        g2 = pltpu.async_copy(hd_h.at[didx.at[j]], hdb, sem1)
                        g1.wait()
                        g2.wait()

                        def edge(e, _):
                            hr = [hsb[e, pl.ds(base + 16 * i, 16)]
                                  for i in range(4)]
                            dr = [hdb[e, pl.ds(base + 16 * i, 16)]
                                  for i in range(4)]
                            sh = []
                            for h in range(2):
                                t0 = hr[2 * h] + dr[2 * h]
                                t1 = hr[2 * h + 1] + dr[2 * h + 1]
                                m0 = jnp.maximum(t0, 0.2 * t0)
                                m1 = jnp.maximum(t1, 0.2 * t1)
                                sh.append(m0 * att[2 * h] + m1 * att[2 * h + 1])
                            eh = [jnp.sum(sh[h], axis=0) for h in range(2)]
                            ev = eh[0] * ohs[0] + eh[1] * ohs[1]
                            exv = jnp.exp(ev)
                            denb[e, :] = jnp.where(hmask, exv, zv)
                            for h in range(2):
                                ex = jnp.sum(jnp.where(iot == h, exv, zv),
                                             axis=0)
                                msgb[e, pl.ds(32 * h, 16)] = hr[2 * h] * ex
                                msgb[e, pl.ds(32 * h + 16, 16)] = hr[2 * h + 1] * ex
                            return 0

                        lax.fori_loop(0, KCH, edge, 0)
                        pltpu.sync_copy(msgb, accn.at[didx.at[j]], add=True)
                        pltpu.sync_copy(denb, accd.at[didx.at[j]], add=True)
                        return 0

                    lax.fori_loop(0, nchunks, chunk, 0)
                    plsc.subcore_barrier()
                    pltpu.sync_copy(accn.at[pl.ds(s * rpt, rpt)],
                                    num_o.at[pl.ds(s * rpt, rpt)])
                    pltpu.sync_copy(accd.at[pl.ds(s * rpt, rpt)],
                                    den_o.at[pl.ds(s * rpt, rpt)])
                    plsc.subcore_barrier()

    return kern


def _pool_kernel():
    @functools.partial(
        pl.kernel,
        out_type=[jax.ShapeDtypeStruct((768, 128), F32),
                  jax.ShapeDtypeStruct((768, 128), F32)],
        mesh=_MESH,
        compiler_params=pltpu.CompilerParams(needs_layout_passes=False,
                                             use_tc_tiling_on_sc=False),
        scratch_types=[
            pltpu.VMEM((128, 128), F32),
            pltpu.VMEM((8, 128), I32),
            pltpu.VMEM_SHARED((768, 128), F32),
        ],
    )
    def kern(x_h, bidx_h, out0, out1, xbuf, bptr, acc):
        c = lax.axis_index("c")
        s = lax.axis_index("s")
        wid = c * 16 + s
        _zero_rows(xbuf, 128, 128)
        pltpu.sync_copy(xbuf.at[pl.ds(0, 48)], acc.at[pl.ds(s * 48, 48)])
        plsc.subcore_barrier()
        pltpu.sync_copy(bidx_h.at[pl.ds(wid * 8, 8)], bptr)

        def chunk(j, _):
            pltpu.sync_copy(x_h.at[pl.ds(wid * 1024 + j * 128, 128)], xbuf)
            pltpu.sync_copy(xbuf, acc.at[bptr.at[j]], add=True)
            return 0

        lax.fori_loop(0, 8, chunk, 0)
        plsc.subcore_barrier()

        @pl.when(c == 0)
        def _():
            pltpu.sync_copy(acc.at[pl.ds(s * 48, 48)],
                            out0.at[pl.ds(s * 48, 48)])

        @pl.when(c == 1)
        def _():
            pltpu.sync_copy(acc.at[pl.ds(s * 48, 48)],
                            out1.at[pl.ds(s * 48, 48)])

    return kern


def _pad_rows(x, n):
    return jnp.concatenate(
        [x, jnp.zeros((n - x.shape[0],) + x.shape[1:], x.dtype)], axis=0)


def _edge_2d(ei, garb):
    e = ei.shape[1]
    t = e // 16
    nchunks = -(-t // KCH)
    chpad = -(-nchunks // 8) * 8
    tp = nchunks * KCH
    pad = 16 * tp - e
    src = jnp.concatenate([ei[0], jnp.zeros((pad,), I32)])
    dst = jnp.concatenate([ei[1], jnp.full((pad,), garb, I32)])
    rows = ((0, 0), (0, chpad - nchunks), (0, 0))
    src = jnp.pad(src.reshape(16, nchunks, KCH), rows).reshape(16 * chpad, KCH)
    dst = jnp.pad(dst.reshape(16, nchunks, KCH), rows,
                  constant_values=garb).reshape(16 * chpad, KCH)
    return src, dst, nchunks


def kernel(x_text, x_audio, x_video, x_z, lin_W, lin_b, conv_Wl, conv_Wr,
           conv_att, conv_bias, proj_W0, proj_b0, proj_W1, proj_b1,
           ei_ta, ei_at, ei_tv, ei_vt, ei_av, ei_va, ei_tz, ei_zt, ei_az,
           ei_za, ei_vz, ei_zv, ei_zz, batch_text, batch_audio, batch_video):
    eis = [ei_ta, ei_at, ei_tv, ei_vt, ei_av, ei_va, ei_tz, ei_zt, ei_az,
           ei_za, ei_vz, ei_zv, ei_zz]

    xstack = jnp.stack([_pad_rows(x_text, NPAD), _pad_rows(x_audio, NPAD),
                        _pad_rows(x_video, NPAD), _pad_rows(x_z, NPAD)])
    segs_p = [jnp.concatenate([b, jnp.full((NPAD - NBIG,), G, I32)])
              for b in (batch_text, batch_audio, batch_video)]
    segf = jnp.stack(segs_p).astype(F32).reshape(3, 20, 512)
    segrow = segf.reshape(60, 1, 512)
    segcol = segf.reshape(60, 512, 1)
    edges = []
    for i, ei in enumerate(eis):
        garb = GARB_Z if _DST_IDS[i] == 3 else GARB_BIG
        edges.append(_edge_2d(ei, garb))

    post = _positions(segrow, segcol)
    lw = jnp.stack([lin_W[0], lin_W[1], lin_W[2], lin_W[0]])
    lb = jnp.broadcast_to(
        jnp.stack([lin_b[0], lin_b[1], lin_b[2], lin_b[0]])[:, None, :],
        (4, 8, 128))
    flags = jnp.asarray([1, 1, 1, 0], I32)
    h = _lin_pe(xstack, lw, lb, post, flags)

    stab = jnp.asarray(_SRC_IDS + _DST_IDS, I32)
    launches = [
        (("at",), ("ta",)),
        (("vt",), ("tv",)),
        (("va",), ("av",)),
        (("zt", "tz"), ("za", "az")),
        (("zv", "vz"), ("zz",)),
    ]
    dst_groups = {"t": ["at", "vt", "zt"], "a": ["ta", "va", "za"],
                  "v": ["tv", "av", "zv"], "z": ["tz", "az", "vz", "zz"]}

    for l in range(2):
        w26 = jnp.concatenate([conv_Wl[l], conv_Wr[l]], axis=0)
        t26 = _project26(stab, h, w26)
        results = {}
        for cfg_jobs in launches:
            cfg = []
            args = []
            names = []
            for core_jobs in cfg_jobs:
                core_cfg = []
                for nm in core_jobs:
                    i = _ENAME2IDX[nm]
                    src2d, dst2d, nchunks = edges[i]
                    nd_acc = ACC_Z if _DST_IDS[i] == 3 else ACC_BIG
                    for half in range(2):
                        core_cfg.append((nchunks, nd_acc, half))
                        args += [t26[i], t26[13 + i], src2d, dst2d,
                                 conv_att[l, i, 2 * half:2 * half + 2]
                                 .reshape(4, 16)]
                        names.append((nm, half))
                cfg.append(tuple(core_cfg))
            outs = _edge_kernel(tuple(cfg))(*args)
            for k, key in enumerate(names):
                results[key] = (outs[2 * k], outs[2 * k + 1])

        news = {}
        for tname, group in dst_groups.items():
            pairs = []
            for nm in group:
                n0, d0 = results[(nm, 0)]
                n1, d1 = results[(nm, 1)]
                num = jnp.concatenate([n0, n1], axis=1)
                den = jnp.pad(
                    jnp.concatenate([d0[:, 0:2], d1[:, 0:2]], axis=1),
                    ((0, 0), (0, 12)))
                pairs.append((num, den))
            bsum = sum(conv_bias[l, _ENAME2IDX[nm]] for nm in group)
            bias8 = jnp.broadcast_to(bsum[None, :], (8, 128))
            nd_acc = ACC_Z if tname == "z" else ACC_BIG
            news[tname] = _combine(nd_acc, pairs, bias8, 1.0 / len(group))
        h = jnp.stack([news["t"], news["a"], news["v"],
                       _pad_rows(news["z"], NPAD)])

    xcat = jnp.concatenate(
        [h[0], h[1], h[2], jnp.zeros((2048, 128), F32)], axis=0)
    bidx = jnp.concatenate(
        segs_p + [jnp.full((2048,), G, I32)]).reshape(256, 128)
    pool0, pool1 = _pool_kernel()(xcat, bidx)
    b0 = jnp.broadcast_to(proj_b0[None, :], (8, 128))
    b1 = jnp.broadcast_to(proj_b1[None, :], (8, 128))
    out = _loss(pool0, pool1, segrow, proj_W0, b0, proj_W1, b1)
    return out[0, 0]


# speedup vs baseline: 24.0059x; 1.0848x over previous
import functools

import jax
import jax.numpy as jnp
import numpy as np
from jax import lax
from jax.experimental import pallas as pl
from jax.experimental.pallas import tpu as pltpu
from jax.experimental.pallas import tpu_sc as plsc

F32 = jnp.float32
I32 = jnp.int32

D = 128
HEADS = 4
CH = 32
G = 512
TEMP = 0.1
NBIG = 10000
NZ = 512
NPAD = 10240
ACC_BIG = 10240
ACC_Z = 640
GARB_BIG = 10000
GARB_Z = 512
KCH = 128
MAXCH = 48

_TID = {"t": 0, "a": 1, "v": 2, "z": 3}
_ETS = [
    ("ta", "t", "a", 90000), ("at", "a", "t", 90000), ("tv", "t", "v", 90000),
    ("vt", "v", "t", 90000), ("av", "a", "v", 90000), ("va", "v", "a", 90000),
    ("tz", "t", "z", 10000), ("zt", "z", "t", 10000), ("az", "a", "z", 10000),
    ("za", "z", "a", 10000), ("vz", "v", "z", 10000), ("zv", "z", "v", 10000),
    ("zz", "z", "z", 4096),
]
_ENAME2IDX = {nm: i for i, (nm, _, _, _) in enumerate(_ETS)}
_SRC_IDS = [_TID[s] for (_, s, _, _) in _ETS]
_DST_IDS = [_TID[d] for (_, _, d, _) in _ETS]

def _pe_consts():
    lane = lax.broadcasted_iota(I32, (1, 128), 1)
    div = jnp.exp(-np.float32(np.log(10000.0) / 128.0)
                  * (lane // 2 * 2).astype(F32))
    even = lane % 2 == 0
    return div, even


def _msel_const():
    row = lax.broadcasted_iota(I32, (16, 128), 0)
    lane = lax.broadcasted_iota(I32, (16, 128), 1)
    return (lane // CH == row).astype(F32)


def _pos_body(srow_ref, scol_ref, o_ref):
    i = pl.program_id(1)
    j = pl.program_id(2)

    @pl.when(j == 0)
    def _():
        o_ref[0] = (lax.broadcasted_iota(I32, (512, 1), 0)
                    + i * 512).astype(F32)

    seg_i = scol_ref[0]
    seg_j = srow_ref[0]
    cmp = (seg_j < seg_i).astype(F32)
    o_ref[0] = o_ref[0] - jnp.sum(cmp, axis=1, keepdims=True)


def _positions(segrow, segcol):
    return pl.pallas_call(
        _pos_body,
        grid=(3, 20, 20),
        in_specs=[
            pl.BlockSpec((1, 1, 512), lambda m, i, j: (m * 20 + j, 0, 0)),
            pl.BlockSpec((1, 512, 1), lambda m, i, j: (m * 20 + i, 0, 0)),
        ],
        out_specs=pl.BlockSpec((1, 512, 1), lambda m, i, j: (m * 20 + i, 0, 0)),
        out_shape=jax.ShapeDtypeStruct((60, 512, 1), F32),
    )(segrow, segcol)


def _lin_body(flag_ref, x_ref, w_ref, b_ref, p_ref, o_ref):
    j = pl.program_id(0)
    h = jnp.dot(x_ref[0], w_ref[0], preferred_element_type=F32) + b_ref[0, 0:1, :]
    pos = p_ref[0]
    div, even = _pe_consts()
    ang = pos * div
    pe = jnp.where(even, jnp.sin(ang), jnp.cos(ang))
    o_ref[0] = h + pe * flag_ref[j].astype(F32)


def _lin_pe(xstack, lw, lb, post, flags):
    return pl.pallas_call(
        _lin_body,
        grid_spec=pltpu.PrefetchScalarGridSpec(
            num_scalar_prefetch=1,
            grid=(4, 20),
            in_specs=[
                pl.BlockSpec((1, 512, 128), lambda j, b, f: (j, b, 0)),
                pl.BlockSpec((1, 128, 128), lambda j, b, f: (j, 0, 0)),
                pl.BlockSpec((1, 8, 128), lambda j, b, f: (j, 0, 0)),
                pl.BlockSpec((1, 512, 1),
                             lambda j, b, f: (jnp.minimum(j * 20 + b, 59), 0, 0)),
            ],
            out_specs=pl.BlockSpec((1, 512, 128), lambda j, b, f: (j, b, 0)),
        ),
        out_shape=jax.ShapeDtypeStruct((4, NPAD, 128), F32),
    )(flags, xstack, lw, lb, post)


def _mm_body(stab_ref, h_ref, w_ref, o_ref):
    o_ref[0] = jnp.dot(h_ref[0], w_ref[0], preferred_element_type=F32)


def _project26(stab, hstack, w26h):
    return pl.pallas_call(
        _mm_body,
        grid_spec=pltpu.PrefetchScalarGridSpec(
            num_scalar_prefetch=1,
            grid=(26, 20),
            in_specs=[
                pl.BlockSpec((1, 512, 128), lambda j, b, s: (s[j], b, 0)),
                pl.BlockSpec((1, 128, 64), lambda j, b, s: (j, 0, 0)),
            ],
            out_specs=pl.BlockSpec((1, 512, 64), lambda j, b, s: (j, b, 0)),
        ),
        out_shape=jax.ShapeDtypeStruct((26, NPAD, 64), F32),
    )(stab, hstack, w26h)


def _combine(nds, pairs, bias_sum, inv_k):
    npairs = len(pairs)

    def body(*refs):
        b_ref = refs[2 * npairs]
        o_ref = refs[2 * npairs + 1]
        msel = _msel_const()
        acc = None
        for p in range(npairs):
            num = refs[2 * p][...]
            den = refs[2 * p + 1][...]
            inv = 1.0 / (den + 1e-16)
            o = num * jnp.dot(inv, msel, preferred_element_type=F32)
            acc = o if acc is None else acc + o
        o_ref[...] = jnp.maximum((acc + b_ref[0:1, :]) * inv_k, 0.0)

    ins = []
    specs = []
    for num, den in pairs:
        ins += [num, den]
        specs += [pl.BlockSpec((128, 128), lambda b: (b, 0)),
                  pl.BlockSpec((128, 16), lambda b: (b, 0))]
    ins.append(bias_sum)
    specs.append(pl.BlockSpec((8, 128), lambda b: (0, 0)))
    return pl.pallas_call(
        body,
        grid=(nds // 128,),
        in_specs=specs,
        out_specs=pl.BlockSpec((128, 128), lambda b: (b, 0)),
        out_shape=jax.ShapeDtypeStruct((nds, 128), F32),
    )(*ins)


def _loss_body(p0_ref, p1_ref, segs_ref, w0_ref, b0_ref, w1_ref, b1_ref, o_ref):
    gid = lax.broadcasted_iota(I32, (512, 1), 0).astype(F32)
    cnt = jnp.zeros((512, 1), F32)
    for r in range(60):
        seg_r = segs_ref[r]
        cnt = cnt + jnp.sum((seg_r == gid).astype(F32), axis=1, keepdims=True)
    s = (p0_ref[0:512, :] + p1_ref[0:512, :]) / jnp.maximum(cnt, 1.0)
    x = jnp.maximum(jnp.dot(s, w0_ref[...], preferred_element_type=F32)
                    + b0_ref[0:1, :], 0.0)
    x = jnp.maximum(jnp.dot(x, w1_ref[...], preferred_element_type=F32)
                    + b1_ref[0:1, :], 0.0)
    nrm = jnp.sqrt(jnp.sum(x * x, axis=1, keepdims=True))
    p = x / jnp.maximum(nrm, 1e-12)
    s_mat = lax.dot_general(p, p, (((1,), (1,)), ((), ())),
                            preferred_element_type=F32) * (1.0 / TEMP)
    eye = (lax.broadcasted_iota(I32, (512, 512), 0)
           == lax.broadcasted_iota(I32, (512, 512), 1)).astype(F32)
    masked = s_mat - eye * 1e9
    m = jnp.max(s_mat, axis=1, keepdims=True)
    lse = m + jnp.log(jnp.sum(jnp.exp(s_mat - m), axis=1, keepdims=True)
                      + jnp.sum(jnp.exp(masked - m), axis=1, keepdims=True))
    diag = jnp.sum(s_mat * eye, axis=1, keepdims=True)
    la = lse - diag
    loss = 2.0 * jnp.sum(la) / 512.0
    o_ref[...] = jnp.full((8, 128), loss, F32)


def _loss(pool0, pool1, segs, w0, b0, w1, b1):
    return pl.pallas_call(
        _loss_body,
        out_shape=jax.ShapeDtypeStruct((8, 128), F32),
    )(pool0, pool1, segs, w0, b0, w1, b1)


_MESH = plsc.VectorSubcoreMesh(core_axis_name="c", subcore_axis_name="s",
                               num_cores=2, num_subcores=16)


def _zero_rows(ref, nrows, width):
    zv = jnp.zeros((16,), F32)

    def bd(r, _):
        for i in range(width // 16):
            ref[r, pl.ds(16 * i, 16)] = zv
        return 0

    lax.fori_loop(0, nrows, bd, 0)


@functools.lru_cache(maxsize=None)
def _edge_kernel(cfg):
    njobs = sum(len(jobs) for jobs in cfg)
    out_type = []
    for jobs in cfg:
        for (_, nd_acc) in jobs:
            out_type.append(jax.ShapeDtypeStruct((nd_acc, 64), F32))
            out_type.append(jax.ShapeDtypeStruct((nd_acc, 16), F32))

    @functools.partial(
        pl.kernel,
        out_type=out_type,
        mesh=_MESH,
        compiler_params=pltpu.CompilerParams(needs_layout_passes=False,
                                             use_tc_tiling_on_sc=False),
        scratch_types=[
            pltpu.VMEM((MAXCH, KCH), I32),
            pltpu.VMEM((MAXCH, KCH), I32),
            pltpu.VMEM((KCH, 64), F32),
            pltpu.VMEM((KCH, 64), F32),
            pltpu.VMEM((KCH, 64), F32),
            pltpu.VMEM((KCH, 64), F32),
            pltpu.VMEM((KCH, 64), F32),
            pltpu.VMEM((KCH, 16), F32),
            pltpu.VMEM((4, 16), F32),
            pltpu.VMEM_SHARED((ACC_BIG, 64), F32),
            pltpu.VMEM_SHARED((ACC_BIG, 16), F32),
            pltpu.SemaphoreType.DMA,
            pltpu.SemaphoreType.DMA,
            pltpu.SemaphoreType.DMA,
            pltpu.SemaphoreType.DMA,
        ],
    )
    def kern(*refs):
        ins = refs[:5 * njobs]
        outs = refs[5 * njobs:5 * njobs + 2 * njobs]
        (sidx, didx, hsb0, hdb0, hsb1, hdb1, msgb, denb, attv,
         accn, accd, sem0, sem1, sem2, sem3) = refs[5 * njobs + 2 * njobs:]
        c = lax.axis_index("c")
        s = lax.axis_index("s")
        iot = lax.iota(I32, 16)
        ohs = [(iot == h).astype(F32) for h in range(2)]
        hmask = iot < 2
        zv = jnp.zeros((16,), F32)

        flat = 0
        for core_id, jobs in enumerate(cfg):
            job_in = ins[5 * flat:]
            job_out = outs[2 * flat:]
            flat += len(jobs)

            @pl.when(c == core_id)
            def _(jobs=jobs, job_in=job_in, job_out=job_out):
                for ji, (nchunks, nd_acc) in enumerate(jobs):
                    hs_h, hd_h, src_h, dst_h, att_h = job_in[5 * ji:5 * ji + 5]
                    num_o, den_o = job_out[2 * ji:2 * ji + 2]
                    rpt = nd_acc // 16
                    _zero_rows(msgb, KCH, 64)
                    _zero_rows(denb, KCH, 16)
                    nfull, rem = rpt // KCH, rpt % KCH
                    for q in range(nfull):
                        pltpu.sync_copy(
                            msgb, accn.at[pl.ds(s * rpt + q * KCH, KCH)])
                        pltpu.sync_copy(
                            denb, accd.at[pl.ds(s * rpt + q * KCH, KCH)])
                    if rem:
                        pltpu.sync_copy(
                            msgb.at[pl.ds(0, rem)],
                            accn.at[pl.ds(s * rpt + nfull * KCH, rem)])
                        pltpu.sync_copy(
                            denb.at[pl.ds(0, rem)],
                            accd.at[pl.ds(s * rpt + nfull * KCH, rem)])
                    plsc.subcore_barrier()
                    chpad = -(-nchunks // 8) * 8
                    pltpu.sync_copy(src_h.at[pl.ds(s * chpad, chpad)],
                                    sidx.at[pl.ds(0, chpad)])
                    pltpu.sync_copy(dst_h.at[pl.ds(s * chpad, chpad)],
                                    didx.at[pl.ds(0, chpad)])
                    pltpu.sync_copy(att_h, attv)
                    att = [attv[i, :] for i in range(4)]

                    def compute_scatter(hsb, hdb, j):
                        def edge(e, _):
                            hr = [hsb[e, pl.ds(16 * i, 16)] for i in range(4)]
                            dr = [hdb[e, pl.ds(16 * i, 16)] for i in range(4)]
                            sh = []
                            for h in range(2):
                                t0 = hr[2 * h] + dr[2 * h]
                                t1 = hr[2 * h + 1] + dr[2 * h + 1]
                                m0 = jnp.maximum(t0, 0.2 * t0)
                                m1 = jnp.maximum(t1, 0.2 * t1)
                                sh.append(m0 * att[2 * h] + m1 * att[2 * h + 1])
                            eh = [jnp.sum(sh[h], axis=0) for h in range(2)]
                            ev = eh[0] * ohs[0] + eh[1] * ohs[1]
                            exv = jnp.exp(ev)
                            denb[e, :] = jnp.where(hmask, exv, zv)
                            for h in range(2):
                                ex = jnp.sum(jnp.where(iot == h, exv, zv),
                                             axis=0)
                                msgb[e, pl.ds(32 * h, 16)] = hr[2 * h] * ex
                                msgb[e, pl.ds(32 * h + 16, 16)] = hr[2 * h + 1] * ex
                            return 0

                        lax.fori_loop(0, KCH, edge, 0)
                        pltpu.sync_copy(msgb, accn.at[didx.at[j]], add=True)
                        pltpu.sync_copy(denb, accd.at[didx.at[j]], add=True)

                    pltpu.async_copy(hs_h.at[sidx.at[0]], hsb0, sem0)
                    pltpu.async_copy(hd_h.at[didx.at[0]], hdb0, sem1)

                    def chunk2(j, _):
                        c0 = 2 * j
                        c1 = 2 * j + 1
                        pltpu.make_async_copy(
                            hs_h.at[sidx.at[c0]], hsb0, sem0).wait()
                        pltpu.make_async_copy(
                            hd_h.at[didx.at[c0]], hdb0, sem1).wait()
                        pltpu.async_copy(hs_h.at[sidx.at[c1]], hsb1, sem2)
                        pltpu.async_copy(hd_h.at[didx.at[c1]], hdb1, sem3)
                        compute_scatter(hsb0, hdb0, c0)
                        pltpu.make_async_copy(
                            hs_h.at[sidx.at[c1]], hsb1, sem2).wait()
                        pltpu.make_async_copy(
                            hd_h.at[didx.at[c1]], hdb1, sem3).wait()

                        @pl.when(c0 + 2 < nchunks)
                        def _():
                            pltpu.async_copy(
                                hs_h.at[sidx.at[c0 + 2]], hsb0, sem0)
                            pltpu.async_copy(
                                hd_h.at[didx.at[c0 + 2]], hdb0, sem1)

                        compute_scatter(hsb1, hdb1, c1)
                        return 0

                    lax.fori_loop(0, nchunks // 2, chunk2, 0)
                    plsc.subcore_barrier()
                    pltpu.sync_copy(accn.at[pl.ds(s * rpt, rpt)],
                                    num_o.at[pl.ds(s * rpt, rpt)])
                    pltpu.sync_copy(accd.at[pl.ds(s * rpt, rpt)],
                                    den_o.at[pl.ds(s * rpt, rpt)])
                    plsc.subcore_barrier()

    return kern


def _pool_kernel():
    @functools.partial(
        pl.kernel,
        out_type=[jax.ShapeDtypeStruct((768, 128), F32),
                  jax.ShapeDtypeStruct((768, 128), F32)],
        mesh=_MESH,
        compiler_params=pltpu.CompilerParams(needs_layout_passes=False,
                                             use_tc_tiling_on_sc=False),
        scratch_types=[
            pltpu.VMEM((128, 128), F32),
            pltpu.VMEM((8, 128), I32),
            pltpu.VMEM_SHARED((768, 128), F32),
        ],
    )
    def kern(x_h, bidx_h, out0, out1, xbuf, bptr, acc):
        c = lax.axis_index("c")
        s = lax.axis_index("s")
        wid = c * 16 + s
        _zero_rows(xbuf, 128, 128)
        pltpu.sync_copy(xbuf.at[pl.ds(0, 48)], acc.at[pl.ds(s * 48, 48)])
        plsc.subcore_barrier()
        pltpu.sync_copy(bidx_h.at[pl.ds(wid * 8, 8)], bptr)

        def chunk(j, _):
            pltpu.sync_copy(x_h.at[pl.ds(wid * 1024 + j * 128, 128)], xbuf)
            pltpu.sync_copy(xbuf, acc.at[bptr.at[j]], add=True)
            return 0

        lax.fori_loop(0, 8, chunk, 0)
        plsc.subcore_barrier()

        @pl.when(c == 0)
        def _():
            pltpu.sync_copy(acc.at[pl.ds(s * 48, 48)],
                            out0.at[pl.ds(s * 48, 48)])

        @pl.when(c == 1)
        def _():
            pltpu.sync_copy(acc.at[pl.ds(s * 48, 48)],
                            out1.at[pl.ds(s * 48, 48)])

    return kern


def _pad_rows(x, n):
    return jnp.concatenate(
        [x, jnp.zeros((n - x.shape[0],) + x.shape[1:], x.dtype)], axis=0)


def _edge_2d(ei, garb):
    e = ei.shape[1]
    t = e // 16
    nchunks = -(-t // KCH)
    nchunks += nchunks % 2
    chpad = -(-nchunks // 8) * 8
    tp = nchunks * KCH
    pad = 16 * tp - e
    src = jnp.concatenate([ei[0], jnp.zeros((pad,), I32)])
    dst = jnp.concatenate([ei[1], jnp.full((pad,), garb, I32)])
    rows = ((0, 0), (0, chpad - nchunks), (0, 0))
    src = jnp.pad(src.reshape(16, nchunks, KCH), rows).reshape(16 * chpad, KCH)
    dst = jnp.pad(dst.reshape(16, nchunks, KCH), rows,
                  constant_values=garb).reshape(16 * chpad, KCH)
    return src, dst, nchunks


def kernel(x_text, x_audio, x_video, x_z, lin_W, lin_b, conv_Wl, conv_Wr,
           conv_att, conv_bias, proj_W0, proj_b0, proj_W1, proj_b1,
           ei_ta, ei_at, ei_tv, ei_vt, ei_av, ei_va, ei_tz, ei_zt, ei_az,
           ei_za, ei_vz, ei_zv, ei_zz, batch_text, batch_audio, batch_video):
    eis = [ei_ta, ei_at, ei_tv, ei_vt, ei_av, ei_va, ei_tz, ei_zt, ei_az,
           ei_za, ei_vz, ei_zv, ei_zz]

    xstack = jnp.stack([_pad_rows(x_text, NPAD), _pad_rows(x_audio, NPAD),
                        _pad_rows(x_video, NPAD), _pad_rows(x_z, NPAD)])
    segs_p = [jnp.concatenate([b, jnp.full((NPAD - NBIG,), G, I32)])
              for b in (batch_text, batch_audio, batch_video)]
    segf = jnp.stack(segs_p).astype(F32).reshape(3, 20, 512)
    segrow = segf.reshape(60, 1, 512)
    segcol = segf.reshape(60, 512, 1)
    edges = []
    for i, ei in enumerate(eis):
        garb = GARB_Z if _DST_IDS[i] == 3 else GARB_BIG
        edges.append(_edge_2d(ei, garb))

    post = _positions(segrow, segcol)
    lw = jnp.stack([lin_W[0], lin_W[1], lin_W[2], lin_W[0]])
    lb = jnp.broadcast_to(
        jnp.stack([lin_b[0], lin_b[1], lin_b[2], lin_b[0]])[:, None, :],
        (4, 8, 128))
    flags = jnp.asarray([1, 1, 1, 0], I32)
    h = _lin_pe(xstack, lw, lb, post, flags)

    stab = jnp.asarray(_SRC_IDS + _DST_IDS, I32)
    launches = [
        (("at",), ("ta",)),
        (("vt",), ("tv",)),
        (("va",), ("av",)),
        (("zt", "tz"), ("za", "az")),
        (("zv", "vz"), ("zz",)),
    ]
    dst_groups = {"t": ["at", "vt", "zt"], "a": ["ta", "va", "za"],
                  "v": ["tv", "av", "zv"], "z": ["tz", "az", "vz", "zz"]}

    for l in range(2):
        w26 = jnp.concatenate([conv_Wl[l], conv_Wr[l]], axis=0)
        t26h = [_project26(stab, h, w26[:, :, :64]),
                _project26(stab, h, w26[:, :, 64:])]
        results = {}
        for cfg_jobs in launches:
            cfg = []
            args = []
            names = []
            for core_jobs in cfg_jobs:
                core_cfg = []
                for nm in core_jobs:
                    i = _ENAME2IDX[nm]
                    src2d, dst2d, nchunks = edges[i]
                    nd_acc = ACC_Z if _DST_IDS[i] == 3 else ACC_BIG
                    for half in range(2):
                        core_cfg.append((nchunks, nd_acc))
                        args += [t26h[half][i], t26h[half][13 + i],
                                 src2d, dst2d,
                                 conv_att[l, i, 2 * half:2 * half + 2]
                                 .reshape(4, 16)]
                        names.append((nm, half))
                cfg.append(tuple(core_cfg))
            outs = _edge_kernel(tuple(cfg))(*args)
            for k, key in enumerate(names):
                results[key] = (outs[2 * k], outs[2 * k + 1])

        news = {}
        for tname, group in dst_groups.items():
            pairs = []
            for nm in group:
                n0, d0 = results[(nm, 0)]
                n1, d1 = results[(nm, 1)]
                num = jnp.concatenate([n0, n1], axis=1)
                den = jnp.pad(
                    jnp.concatenate([d0[:, 0:2], d1[:, 0:2]], axis=1),
                    ((0, 0), (0, 12)))
                pairs.append((num, den))
            bsum = sum(conv_bias[l, _ENAME2IDX[nm]] for nm in group)
            bias8 = jnp.broadcast_to(bsum[None, :], (8, 128))
            nd_acc = ACC_Z if tname == "z" else ACC_BIG
            news[tname] = _combine(nd_acc, pairs, bias8, 1.0 / len(group))
        h = jnp.stack([news["t"], news["a"], news["v"],
                       _pad_rows(news["z"], NPAD)])

    xcat = jnp.concatenate(
        [h[0], h[1], h[2], jnp.zeros((2048, 128), F32)], axis=0)
    bidx = jnp.concatenate(
        segs_p + [jnp.full((2048,), G, I32)]).reshape(256, 128)
    pool0, pool1 = _pool_kernel()(xcat, bidx)
    b0 = jnp.broadcast_to(proj_b0[None, :], (8, 128))
    b1 = jnp.broadcast_to(proj_b1[None, :], (8, 128))
    out = _loss(pool0, pool1, segrow, proj_W0, b0, proj_W1, b1)
    return out[0, 0]


# speedup vs baseline: 24.4804x; 1.0198x over previous
import functools

import jax
import jax.numpy as jnp
import numpy as np
from jax import lax
from jax.experimental import pallas as pl
from jax.experimental.pallas import tpu as pltpu
from jax.experimental.pallas import tpu_sc as plsc

F32 = jnp.float32
I32 = jnp.int32

D = 128
HEADS = 4
CH = 32
G = 512
TEMP = 0.1
NBIG = 10000
NZ = 512
NPAD = 10240
ACC_BIG = 10240
ACC_Z = 640
GARB_BIG = 10000
GARB_Z = 512
KCH = 128
MAXCH = 48

_TID = {"t": 0, "a": 1, "v": 2, "z": 3}
_ETS = [
    ("ta", "t", "a", 90000), ("at", "a", "t", 90000), ("tv", "t", "v", 90000),
    ("vt", "v", "t", 90000), ("av", "a", "v", 90000), ("va", "v", "a", 90000),
    ("tz", "t", "z", 10000), ("zt", "z", "t", 10000), ("az", "a", "z", 10000),
    ("za", "z", "a", 10000), ("vz", "v", "z", 10000), ("zv", "z", "v", 10000),
    ("zz", "z", "z", 4096),
]
_ENAME2IDX = {nm: i for i, (nm, _, _, _) in enumerate(_ETS)}
_SRC_IDS = [_TID[s] for (_, s, _, _) in _ETS]
_DST_IDS = [_TID[d] for (_, _, d, _) in _ETS]

def _pe_consts():
    lane = lax.broadcasted_iota(I32, (1, 128), 1)
    div = jnp.exp(-np.float32(np.log(10000.0) / 128.0)
                  * (lane // 2 * 2).astype(F32))
    even = lane % 2 == 0
    return div, even


def _msel_const():
    row = lax.broadcasted_iota(I32, (16, 128), 0)
    lane = lax.broadcasted_iota(I32, (16, 128), 1)
    return (lane // CH == row).astype(F32)


def _pos_body(srow_ref, scol_ref, o_ref):
    i = pl.program_id(1)
    j = pl.program_id(2)

    @pl.when(j == 0)
    def _():
        o_ref[0] = (lax.broadcasted_iota(I32, (512, 1), 0)
                    + i * 512).astype(F32)

    seg_i = scol_ref[0]
    seg_j = srow_ref[0]
    cmp = (seg_j < seg_i).astype(F32)
    o_ref[0] = o_ref[0] - jnp.sum(cmp, axis=1, keepdims=True)


def _positions(segrow, segcol):
    return pl.pallas_call(
        _pos_body,
        grid=(3, 20, 20),
        in_specs=[
            pl.BlockSpec((1, 1, 512), lambda m, i, j: (m * 20 + j, 0, 0)),
            pl.BlockSpec((1, 512, 1), lambda m, i, j: (m * 20 + i, 0, 0)),
        ],
        out_specs=pl.BlockSpec((1, 512, 1), lambda m, i, j: (m * 20 + i, 0, 0)),
        out_shape=jax.ShapeDtypeStruct((60, 512, 1), F32),
    )(segrow, segcol)


def _lin_body(flag_ref, x_ref, w_ref, b_ref, p_ref, o_ref):
    j = pl.program_id(0)
    h = jnp.dot(x_ref[0], w_ref[0], preferred_element_type=F32) + b_ref[0, 0:1, :]
    pos = p_ref[0]
    div, even = _pe_consts()
    ang = pos * div
    pe = jnp.where(even, jnp.sin(ang), jnp.cos(ang))
    o_ref[0] = h + pe * flag_ref[j].astype(F32)


def _lin_pe(xstack, lw, lb, post, flags):
    return pl.pallas_call(
        _lin_body,
        grid_spec=pltpu.PrefetchScalarGridSpec(
            num_scalar_prefetch=1,
            grid=(4, 20),
            in_specs=[
                pl.BlockSpec((1, 512, 128), lambda j, b, f: (j, b, 0)),
                pl.BlockSpec((1, 128, 128), lambda j, b, f: (j, 0, 0)),
                pl.BlockSpec((1, 8, 128), lambda j, b, f: (j, 0, 0)),
                pl.BlockSpec((1, 512, 1),
                             lambda j, b, f: (jnp.minimum(j * 20 + b, 59), 0, 0)),
            ],
            out_specs=pl.BlockSpec((1, 512, 128), lambda j, b, f: (j, b, 0)),
        ),
        out_shape=jax.ShapeDtypeStruct((4, NPAD, 128), F32),
    )(flags, xstack, lw, lb, post)


def _mm_body(stab_ref, h_ref, w_ref, o_ref):
    o_ref[0] = jnp.dot(h_ref[0], w_ref[0], preferred_element_type=F32)


def _project26(stab, hstack, w26h):
    return pl.pallas_call(
        _mm_body,
        grid_spec=pltpu.PrefetchScalarGridSpec(
            num_scalar_prefetch=1,
            grid=(26, 20),
            in_specs=[
                pl.BlockSpec((1, 512, 128), lambda j, b, s: (s[j], b, 0)),
                pl.BlockSpec((1, 128, 64), lambda j, b, s: (j, 0, 0)),
            ],
            out_specs=pl.BlockSpec((1, 512, 64), lambda j, b, s: (j, b, 0)),
        ),
        out_shape=jax.ShapeDtypeStruct((26, NPAD, 64), F32),
    )(stab, hstack, w26h)


def _combine(nds, pairs, bias_sum, inv_k):
    npairs = len(pairs)

    def body(*refs):
        b_ref = refs[2 * npairs]
        o_ref = refs[2 * npairs + 1]
        msel = _msel_const()
        acc = None
        for p in range(npairs):
            num = refs[2 * p][...]
            den = refs[2 * p + 1][...]
            inv = 1.0 / (den + 1e-16)
            o = num * jnp.dot(inv, msel, preferred_element_type=F32)
            acc = o if acc is None else acc + o
        o_ref[...] = jnp.maximum((acc + b_ref[0:1, :]) * inv_k, 0.0)

    ins = []
    specs = []
    for num, den in pairs:
        ins += [num, den]
        specs += [pl.BlockSpec((128, 128), lambda b: (b, 0)),
                  pl.BlockSpec((128, 16), lambda b: (b, 0))]
    ins.append(bias_sum)
    specs.append(pl.BlockSpec((8, 128), lambda b: (0, 0)))
    return pl.pallas_call(
        body,
        grid=(nds // 128,),
        in_specs=specs,
        out_specs=pl.BlockSpec((128, 128), lambda b: (b, 0)),
        out_shape=jax.ShapeDtypeStruct((nds, 128), F32),
    )(*ins)


def _loss_body(p0_ref, p1_ref, segs_ref, w0_ref, b0_ref, w1_ref, b1_ref, o_ref):
    gid = lax.broadcasted_iota(I32, (512, 1), 0).astype(F32)
    cnt = jnp.zeros((512, 1), F32)
    for r in range(60):
        seg_r = segs_ref[r]
        cnt = cnt + jnp.sum((seg_r == gid).astype(F32), axis=1, keepdims=True)
    s = (p0_ref[0:512, :] + p1_ref[0:512, :]) / jnp.maximum(cnt, 1.0)
    x = jnp.maximum(jnp.dot(s, w0_ref[...], preferred_element_type=F32)
                    + b0_ref[0:1, :], 0.0)
    x = jnp.maximum(jnp.dot(x, w1_ref[...], preferred_element_type=F32)
                    + b1_ref[0:1, :], 0.0)
    nrm = jnp.sqrt(jnp.sum(x * x, axis=1, keepdims=True))
    p = x / jnp.maximum(nrm, 1e-12)
    s_mat = lax.dot_general(p, p, (((1,), (1,)), ((), ())),
                            preferred_element_type=F32) * (1.0 / TEMP)
    eye = (lax.broadcasted_iota(I32, (512, 512), 0)
           == lax.broadcasted_iota(I32, (512, 512), 1)).astype(F32)
    masked = s_mat - eye * 1e9
    m = jnp.max(s_mat, axis=1, keepdims=True)
    lse = m + jnp.log(jnp.sum(jnp.exp(s_mat - m), axis=1, keepdims=True)
                      + jnp.sum(jnp.exp(masked - m), axis=1, keepdims=True))
    diag = jnp.sum(s_mat * eye, axis=1, keepdims=True)
    la = lse - diag
    loss = 2.0 * jnp.sum(la) / 512.0
    o_ref[...] = jnp.full((8, 128), loss, F32)


def _loss(pool0, pool1, segs, w0, b0, w1, b1):
    return pl.pallas_call(
        _loss_body,
        out_shape=jax.ShapeDtypeStruct((8, 128), F32),
    )(pool0, pool1, segs, w0, b0, w1, b1)


_MESH = plsc.VectorSubcoreMesh(core_axis_name="c", subcore_axis_name="s",
                               num_cores=2, num_subcores=16)


def _zero_rows(ref, nrows, width):
    zv = jnp.zeros((16,), F32)

    def bd(r, _):
        for i in range(width // 16):
            ref[r, pl.ds(16 * i, 16)] = zv
        return 0

    lax.fori_loop(0, nrows, bd, 0)


@functools.lru_cache(maxsize=None)
def _edge_kernel(cfg):
    njobs = sum(len(jobs) for jobs in cfg)
    out_type = []
    for jobs in cfg:
        for (_, nd_acc) in jobs:
            out_type.append(jax.ShapeDtypeStruct((nd_acc, 64), F32))
            out_type.append(jax.ShapeDtypeStruct((nd_acc, 16), F32))

    @functools.partial(
        pl.kernel,
        out_type=out_type,
        mesh=_MESH,
        compiler_params=pltpu.CompilerParams(needs_layout_passes=False,
                                             use_tc_tiling_on_sc=False),
        scratch_types=[
            pltpu.VMEM((MAXCH, KCH), I32),
            pltpu.VMEM((MAXCH, KCH), I32),
            pltpu.VMEM((KCH, 64), F32),
            pltpu.VMEM((KCH, 64), F32),
            pltpu.VMEM((KCH, 64), F32),
            pltpu.VMEM((KCH, 64), F32),
            pltpu.VMEM((KCH, 64), F32),
            pltpu.VMEM((KCH, 16), F32),
            pltpu.VMEM((4, 16), F32),
            pltpu.VMEM_SHARED((ACC_BIG, 64), F32),
            pltpu.VMEM_SHARED((ACC_BIG, 16), F32),
            pltpu.SemaphoreType.DMA,
            pltpu.SemaphoreType.DMA,
            pltpu.SemaphoreType.DMA,
            pltpu.SemaphoreType.DMA,
        ],
    )
    def kern(*refs):
        ins = refs[:5 * njobs]
        outs = refs[5 * njobs:5 * njobs + 2 * njobs]
        (sidx, didx, hsb0, hdb0, hsb1, hdb1, msgb, denb, attv,
         accn, accd, sem0, sem1, sem2, sem3) = refs[5 * njobs + 2 * njobs:]
        c = lax.axis_index("c")
        s = lax.axis_index("s")
        iot = lax.iota(I32, 16)
        zv = jnp.zeros((16,), F32)
        i15 = jnp.full((16,), 15, I32)

        flat = 0
        for core_id, jobs in enumerate(cfg):
            job_in = ins[5 * flat:]
            job_out = outs[2 * flat:]
            flat += len(jobs)

            @pl.when(c == core_id)
            def _(jobs=jobs, job_in=job_in, job_out=job_out):
                for ji, (nchunks, nd_acc) in enumerate(jobs):
                    hs_h, hd_h, src_h, dst_h, att_h = job_in[5 * ji:5 * ji + 5]
                    num_o, den_o = job_out[2 * ji:2 * ji + 2]
                    rpt = nd_acc // 16
                    _zero_rows(msgb, KCH, 64)
                    _zero_rows(denb, KCH, 16)
                    nfull, rem = rpt // KCH, rpt % KCH
                    for q in range(nfull):
                        pltpu.sync_copy(
                            msgb, accn.at[pl.ds(s * rpt + q * KCH, KCH)])
                        pltpu.sync_copy(
                            denb, accd.at[pl.ds(s * rpt + q * KCH, KCH)])
                    if rem:
                        pltpu.sync_copy(
                            msgb.at[pl.ds(0, rem)],
                            accn.at[pl.ds(s * rpt + nfull * KCH, rem)])
                        pltpu.sync_copy(
                            denb.at[pl.ds(0, rem)],
                            accd.at[pl.ds(s * rpt + nfull * KCH, rem)])
                    plsc.subcore_barrier()
                    chpad = -(-nchunks // 8) * 8
                    pltpu.sync_copy(src_h.at[pl.ds(s * chpad, chpad)],
                                    sidx.at[pl.ds(0, chpad)])
                    pltpu.sync_copy(dst_h.at[pl.ds(s * chpad, chpad)],
                                    didx.at[pl.ds(0, chpad)])
                    pltpu.sync_copy(att_h, attv)
                    att = [attv[i, :] for i in range(4)]

                    def compute_scatter(hsb, hdb, j):
                        def one_edge(e):
                            hr = [hsb[e, pl.ds(16 * i, 16)] for i in range(4)]
                            dr = [hdb[e, pl.ds(16 * i, 16)] for i in range(4)]
                            sh = []
                            for h in range(2):
                                t0 = hr[2 * h] + dr[2 * h]
                                t1 = hr[2 * h + 1] + dr[2 * h + 1]
                                m0 = jnp.maximum(t0, 0.2 * t0)
                                m1 = jnp.maximum(t1, 0.2 * t1)
                                sh.append(m0 * att[2 * h] + m1 * att[2 * h + 1])
                            ex0 = jnp.exp(plsc.cumsum(sh[0])[i15])
                            ex1 = jnp.exp(plsc.cumsum(sh[1])[i15])
                            denb[e, :] = jnp.where(
                                iot == 0, ex0, jnp.where(iot == 1, ex1, zv))
                            msgb[e, pl.ds(0, 16)] = hr[0] * ex0
                            msgb[e, pl.ds(16, 16)] = hr[1] * ex0
                            msgb[e, pl.ds(32, 16)] = hr[2] * ex1
                            msgb[e, pl.ds(48, 16)] = hr[3] * ex1

                        def edge4(k, _):
                            for u in range(4):
                                one_edge(4 * k + u)
                            return 0

                        lax.fori_loop(0, KCH // 4, edge4, 0)
                        pltpu.sync_copy(msgb, accn.at[didx.at[j]], add=True)
                        pltpu.sync_copy(denb, accd.at[didx.at[j]], add=True)

                    pltpu.async_copy(hs_h.at[sidx.at[0]], hsb0, sem0)
                    pltpu.async_copy(hd_h.at[didx.at[0]], hdb0, sem1)

                    def chunk2(j, _):
                        c0 = 2 * j
                        c1 = 2 * j + 1
                        pltpu.make_async_copy(
                            hs_h.at[sidx.at[c0]], hsb0, sem0).wait()
                        pltpu.make_async_copy(
                            hd_h.at[didx.at[c0]], hdb0, sem1).wait()
                        pltpu.async_copy(hs_h.at[sidx.at[c1]], hsb1, sem2)
                        pltpu.async_copy(hd_h.at[didx.at[c1]], hdb1, sem3)
                        compute_scatter(hsb0, hdb0, c0)
                        pltpu.make_async_copy(
                            hs_h.at[sidx.at[c1]], hsb1, sem2).wait()
                        pltpu.make_async_copy(
                            hd_h.at[didx.at[c1]], hdb1, sem3).wait()

                        @pl.when(c0 + 2 < nchunks)
                        def _():
                            pltpu.async_copy(
                                hs_h.at[sidx.at[c0 + 2]], hsb0, sem0)
                            pltpu.async_copy(
                                hd_h.at[didx.at[c0 + 2]], hdb0, sem1)

                        compute_scatter(hsb1, hdb1, c1)
                        return 0

                    lax.fori_loop(0, nchunks // 2, chunk2, 0)
                    plsc.subcore_barrier()
                    pltpu.sync_copy(accn.at[pl.ds(s * rpt, rpt)],
                                    num_o.at[pl.ds(s * rpt, rpt)])
                    pltpu.sync_copy(accd.at[pl.ds(s * rpt, rpt)],
                                    den_o.at[pl.ds(s * rpt, rpt)])
                    plsc.subcore_barrier()

    return kern


def _pool_kernel():
    @functools.partial(
        pl.kernel,
        out_type=[jax.ShapeDtypeStruct((768, 128), F32),
                  jax.ShapeDtypeStruct((768, 128), F32)],
        mesh=_MESH,
        compiler_params=pltpu.CompilerParams(needs_layout_passes=False,
                                             use_tc_tiling_on_sc=False),
        scratch_types=[
            pltpu.VMEM((128, 128), F32),
            pltpu.VMEM((8, 128), I32),
            pltpu.VMEM_SHARED((768, 128), F32),
        ],
    )
    def kern(x_h, bidx_h, out0, out1, xbuf, bptr, acc):
        c = lax.axis_index("c")
        s = lax.axis_index("s")
        wid = c * 16 + s
        _zero_rows(xbuf, 128, 128)
        pltpu.sync_copy(xbuf.at[pl.ds(0, 48)], acc.at[pl.ds(s * 48, 48)])
        plsc.subcore_barrier()
        pltpu.sync_copy(bidx_h.at[pl.ds(wid * 8, 8)], bptr)

        def chunk(j, _):
            pltpu.sync_copy(x_h.at[pl.ds(wid * 1024 + j * 128, 128)], xbuf)
            pltpu.sync_copy(xbuf, acc.at[bptr.at[j]], add=True)
            return 0

        lax.fori_loop(0, 8, chunk, 0)
        plsc.subcore_barrier()

        @pl.when(c == 0)
        def _():
            pltpu.sync_copy(acc.at[pl.ds(s * 48, 48)],
                            out0.at[pl.ds(s * 48, 48)])

        @pl.when(c == 1)
        def _():
            pltpu.sync_copy(acc.at[pl.ds(s * 48, 48)],
                            out1.at[pl.ds(s * 48, 48)])

    return kern


def _pad_rows(x, n):
    return jnp.concatenate(
        [x, jnp.zeros((n - x.shape[0],) + x.shape[1:], x.dtype)], axis=0)


def _edge_2d(ei, garb):
    e = ei.shape[1]
    t = e // 16
    nchunks = -(-t // KCH)
    nchunks += nchunks % 2
    chpad = -(-nchunks // 8) * 8
    tp = nchunks * KCH
    pad = 16 * tp - e
    src = jnp.concatenate([ei[0], jnp.zeros((pad,), I32)])
    dst = jnp.concatenate([ei[1], jnp.full((pad,), garb, I32)])
    rows = ((0, 0), (0, chpad - nchunks), (0, 0))
    src = jnp.pad(src.reshape(16, nchunks, KCH), rows).reshape(16 * chpad, KCH)
    dst = jnp.pad(dst.reshape(16, nchunks, KCH), rows,
                  constant_values=garb).reshape(16 * chpad, KCH)
    return src, dst, nchunks


def kernel(x_text, x_audio, x_video, x_z, lin_W, lin_b, conv_Wl, conv_Wr,
           conv_att, conv_bias, proj_W0, proj_b0, proj_W1, proj_b1,
           ei_ta, ei_at, ei_tv, ei_vt, ei_av, ei_va, ei_tz, ei_zt, ei_az,
           ei_za, ei_vz, ei_zv, ei_zz, batch_text, batch_audio, batch_video):
    eis = [ei_ta, ei_at, ei_tv, ei_vt, ei_av, ei_va, ei_tz, ei_zt, ei_az,
           ei_za, ei_vz, ei_zv, ei_zz]

    xstack = jnp.stack([_pad_rows(x_text, NPAD), _pad_rows(x_audio, NPAD),
                        _pad_rows(x_video, NPAD), _pad_rows(x_z, NPAD)])
    segs_p = [jnp.concatenate([b, jnp.full((NPAD - NBIG,), G, I32)])
              for b in (batch_text, batch_audio, batch_video)]
    segf = jnp.stack(segs_p).astype(F32).reshape(3, 20, 512)
    segrow = segf.reshape(60, 1, 512)
    segcol = segf.reshape(60, 512, 1)
    edges = []
    for i, ei in enumerate(eis):
        garb = GARB_Z if _DST_IDS[i] == 3 else GARB_BIG
        edges.append(_edge_2d(ei, garb))

    post = _positions(segrow, segcol)
    lw = jnp.stack([lin_W[0], lin_W[1], lin_W[2], lin_W[0]])
    lb = jnp.broadcast_to(
        jnp.stack([lin_b[0], lin_b[1], lin_b[2], lin_b[0]])[:, None, :],
        (4, 8, 128))
    flags = jnp.asarray([1, 1, 1, 0], I32)
    h = _lin_pe(xstack, lw, lb, post, flags)

    stab = jnp.asarray(_SRC_IDS + _DST_IDS, I32)
    launches = [
        (("at",), ("ta",)),
        (("vt",), ("tv",)),
        (("va",), ("av",)),
        (("zt", "tz"), ("za", "az")),
        (("zv", "vz"), ("zz",)),
    ]
    dst_groups = {"t": ["at", "vt", "zt"], "a": ["ta", "va", "za"],
                  "v": ["tv", "av", "zv"], "z": ["tz", "az", "vz", "zz"]}

    for l in range(2):
        w26 = jnp.concatenate([conv_Wl[l], conv_Wr[l]], axis=0)
        t26h = [_project26(stab, h, w26[:, :, :64]),
                _project26(stab, h, w26[:, :, 64:])]
        results = {}
        for cfg_jobs in launches:
            cfg = []
            args = []
            names = []
            for core_jobs in cfg_jobs:
                core_cfg = []
                for nm in core_jobs:
                    i = _ENAME2IDX[nm]
                    src2d, dst2d, nchunks = edges[i]
                    nd_acc = ACC_Z if _DST_IDS[i] == 3 else ACC_BIG
                    for half in range(2):
                        core_cfg.append((nchunks, nd_acc))
                        args += [t26h[half][i], t26h[half][13 + i],
                                 src2d, dst2d,
                                 conv_att[l, i, 2 * half:2 * half + 2]
                                 .reshape(4, 16)]
                        names.append((nm, half))
                cfg.append(tuple(core_cfg))
            outs = _edge_kernel(tuple(cfg))(*args)
            for k, key in enumerate(names):
                results[key] = (outs[2 * k], outs[2 * k + 1])

        news = {}
        for tname, group in dst_groups.items():
            pairs = []
            for nm in group:
                n0, d0 = results[(nm, 0)]
                n1, d1 = results[(nm, 1)]
                num = jnp.concatenate([n0, n1], axis=1)
                den = jnp.pad(
                    jnp.concatenate([d0[:, 0:2], d1[:, 0:2]], axis=1),
                    ((0, 0), (0, 12)))
                pairs.append((num, den))
            bsum = sum(conv_bias[l, _ENAME2IDX[nm]] for nm in group)
            bias8 = jnp.broadcast_to(bsum[None, :], (8, 128))
            nd_acc = ACC_Z if tname == "z" else ACC_BIG
            news[tname] = _combine(nd_acc, pairs, bias8, 1.0 / len(group))
        h = jnp.stack([news["t"], news["a"], news["v"],
                       _pad_rows(news["z"], NPAD)])

    xcat = jnp.concatenate(
        [h[0], h[1], h[2], jnp.zeros((2048, 128), F32)], axis=0)
    bidx = jnp.concatenate(
        segs_p + [jnp.full((2048,), G, I32)]).reshape(256, 128)
    pool0, pool1 = _pool_kernel()(xcat, bidx)
    b0 = jnp.broadcast_to(proj_b0[None, :], (8, 128))
    b1 = jnp.broadcast_to(proj_b1[None, :], (8, 128))
    out = _loss(pool0, pool1, segrow, proj_W0, b0, proj_W1, b1)
    return out[0, 0]


# speedup vs baseline: 25.3956x; 1.0374x over previous
import functools

import jax
import jax.numpy as jnp
import numpy as np
from jax import lax
from jax.experimental import pallas as pl
from jax.experimental.pallas import tpu as pltpu
from jax.experimental.pallas import tpu_sc as plsc

F32 = jnp.float32
I32 = jnp.int32

D = 128
HEADS = 4
CH = 32
G = 512
TEMP = 0.1
NBIG = 10000
NZ = 512
NPAD = 10240
ACC_BIG = 10240
ACC_Z = 640
GARB_BIG = 10000
GARB_Z = 512
KCH = 128
MAXCH = 48

_TID = {"t": 0, "a": 1, "v": 2, "z": 3}
_ETS = [
    ("ta", "t", "a", 90000), ("at", "a", "t", 90000), ("tv", "t", "v", 90000),
    ("vt", "v", "t", 90000), ("av", "a", "v", 90000), ("va", "v", "a", 90000),
    ("tz", "t", "z", 10000), ("zt", "z", "t", 10000), ("az", "a", "z", 10000),
    ("za", "z", "a", 10000), ("vz", "v", "z", 10000), ("zv", "z", "v", 10000),
    ("zz", "z", "z", 4096),
]
_ENAME2IDX = {nm: i for i, (nm, _, _, _) in enumerate(_ETS)}
_SRC_IDS = [_TID[s] for (_, s, _, _) in _ETS]
_DST_IDS = [_TID[d] for (_, _, d, _) in _ETS]

def _pe_consts():
    lane = lax.broadcasted_iota(I32, (1, 128), 1)
    div = jnp.exp(-np.float32(np.log(10000.0) / 128.0)
                  * (lane // 2 * 2).astype(F32))
    even = lane % 2 == 0
    return div, even


def _msel_const():
    row = lax.broadcasted_iota(I32, (16, 128), 0)
    lane = lax.broadcasted_iota(I32, (16, 128), 1)
    return (lane // CH == row).astype(F32)


def _pos_body(srow_ref, scol_ref, o_ref):
    i = pl.program_id(1)
    j = pl.program_id(2)

    @pl.when(j == 0)
    def _():
        o_ref[0] = (lax.broadcasted_iota(I32, (512, 1), 0)
                    + i * 512).astype(F32)

    seg_i = scol_ref[0]
    seg_j = srow_ref[0]
    cmp = (seg_j < seg_i).astype(F32)
    o_ref[0] = o_ref[0] - jnp.sum(cmp, axis=1, keepdims=True)


def _positions(segrow, segcol):
    return pl.pallas_call(
        _pos_body,
        grid=(3, 20, 20),
        in_specs=[
            pl.BlockSpec((1, 1, 512), lambda m, i, j: (m * 20 + j, 0, 0)),
            pl.BlockSpec((1, 512, 1), lambda m, i, j: (m * 20 + i, 0, 0)),
        ],
        out_specs=pl.BlockSpec((1, 512, 1), lambda m, i, j: (m * 20 + i, 0, 0)),
        out_shape=jax.ShapeDtypeStruct((60, 512, 1), F32),
    )(segrow, segcol)


def _lin_body(flag_ref, x_ref, w_ref, b_ref, p_ref, o_ref):
    j = pl.program_id(0)
    h = jnp.dot(x_ref[0], w_ref[0], preferred_element_type=F32) + b_ref[0, 0:1, :]
    pos = p_ref[0]
    div, even = _pe_consts()
    ang = pos * div
    pe = jnp.where(even, jnp.sin(ang), jnp.cos(ang))
    o_ref[0] = h + pe * flag_ref[j].astype(F32)


def _lin_pe(xstack, lw, lb, post, flags):
    return pl.pallas_call(
        _lin_body,
        grid_spec=pltpu.PrefetchScalarGridSpec(
            num_scalar_prefetch=1,
            grid=(4, 20),
            in_specs=[
                pl.BlockSpec((1, 512, 128), lambda j, b, f: (j, b, 0)),
                pl.BlockSpec((1, 128, 128), lambda j, b, f: (j, 0, 0)),
                pl.BlockSpec((1, 8, 128), lambda j, b, f: (j, 0, 0)),
                pl.BlockSpec((1, 512, 1),
                             lambda j, b, f: (jnp.minimum(j * 20 + b, 59), 0, 0)),
            ],
            out_specs=pl.BlockSpec((1, 512, 128), lambda j, b, f: (j, b, 0)),
        ),
        out_shape=jax.ShapeDtypeStruct((4, NPAD, 128), F32),
    )(flags, xstack, lw, lb, post)


def _mm_body(stab_ref, h_ref, w_ref, o_ref):
    o_ref[0] = jnp.dot(h_ref[0], w_ref[0], preferred_element_type=F32)


def _project26(stab, hstack, w26h):
    return pl.pallas_call(
        _mm_body,
        grid_spec=pltpu.PrefetchScalarGridSpec(
            num_scalar_prefetch=1,
            grid=(26, 20),
            in_specs=[
                pl.BlockSpec((1, 512, 128), lambda j, b, s: (s[j], b, 0)),
                pl.BlockSpec((1, 128, 64), lambda j, b, s: (j, 0, 0)),
            ],
            out_specs=pl.BlockSpec((1, 512, 64), lambda j, b, s: (j, b, 0)),
        ),
        out_shape=jax.ShapeDtypeStruct((26, NPAD, 64), F32),
    )(stab, hstack, w26h)


def _combine(nds, pairs, bias_sum, inv_k):
    npairs = len(pairs)

    def body(*refs):
        b_ref = refs[2 * npairs]
        o_ref = refs[2 * npairs + 1]
        msel = _msel_const()
        acc = None
        for p in range(npairs):
            num = refs[2 * p][...]
            den = refs[2 * p + 1][...]
            inv = 1.0 / (den + 1e-16)
            o = num * jnp.dot(inv, msel, preferred_element_type=F32)
            acc = o if acc is None else acc + o
        o_ref[...] = jnp.maximum((acc + b_ref[0:1, :]) * inv_k, 0.0)

    ins = []
    specs = []
    for num, den in pairs:
        ins += [num, den]
        specs += [pl.BlockSpec((128, 128), lambda b: (b, 0)),
                  pl.BlockSpec((128, 16), lambda b: (b, 0))]
    ins.append(bias_sum)
    specs.append(pl.BlockSpec((8, 128), lambda b: (0, 0)))
    return pl.pallas_call(
        body,
        grid=(nds // 128,),
        in_specs=specs,
        out_specs=pl.BlockSpec((128, 128), lambda b: (b, 0)),
        out_shape=jax.ShapeDtypeStruct((nds, 128), F32),
    )(*ins)


def _loss_body(p0_ref, p1_ref, segs_ref, w0_ref, b0_ref, w1_ref, b1_ref, o_ref):
    gid = lax.broadcasted_iota(I32, (512, 1), 0).astype(F32)
    cnt = jnp.zeros((512, 1), F32)
    for r in range(60):
        seg_r = segs_ref[r]
        cnt = cnt + jnp.sum((seg_r == gid).astype(F32), axis=1, keepdims=True)
    s = (p0_ref[0:512, :] + p1_ref[0:512, :]) / jnp.maximum(cnt, 1.0)
    x = jnp.maximum(jnp.dot(s, w0_ref[...], preferred_element_type=F32)
                    + b0_ref[0:1, :], 0.0)
    x = jnp.maximum(jnp.dot(x, w1_ref[...], preferred_element_type=F32)
                    + b1_ref[0:1, :], 0.0)
    nrm = jnp.sqrt(jnp.sum(x * x, axis=1, keepdims=True))
    p = x / jnp.maximum(nrm, 1e-12)
    s_mat = lax.dot_general(p, p, (((1,), (1,)), ((), ())),
                            preferred_element_type=F32) * (1.0 / TEMP)
    eye = (lax.broadcasted_iota(I32, (512, 512), 0)
           == lax.broadcasted_iota(I32, (512, 512), 1)).astype(F32)
    masked = s_mat - eye * 1e9
    m = jnp.max(s_mat, axis=1, keepdims=True)
    lse = m + jnp.log(jnp.sum(jnp.exp(s_mat - m), axis=1, keepdims=True)
                      + jnp.sum(jnp.exp(masked - m), axis=1, keepdims=True))
    diag = jnp.sum(s_mat * eye, axis=1, keepdims=True)
    la = lse - diag
    loss = 2.0 * jnp.sum(la) / 512.0
    o_ref[...] = jnp.full((8, 128), loss, F32)


def _loss(pool0, pool1, segs, w0, b0, w1, b1):
    return pl.pallas_call(
        _loss_body,
        out_shape=jax.ShapeDtypeStruct((8, 128), F32),
    )(pool0, pool1, segs, w0, b0, w1, b1)


_MESH = plsc.VectorSubcoreMesh(core_axis_name="c", subcore_axis_name="s",
                               num_cores=2, num_subcores=16)


def _zero_rows(ref, nrows, width):
    zv = jnp.zeros((16,), F32)

    def bd(r, _):
        for i in range(width // 16):
            ref[r, pl.ds(16 * i, 16)] = zv
        return 0

    lax.fori_loop(0, nrows, bd, 0)


@functools.lru_cache(maxsize=None)
def _edge_kernel(cfg):
    njobs = sum(len(jobs) for jobs in cfg)
    out_type = []
    for jobs in cfg:
        for (_, nd_acc) in jobs:
            out_type.append(jax.ShapeDtypeStruct((nd_acc, 64), F32))
            out_type.append(jax.ShapeDtypeStruct((nd_acc, 16), F32))

    @functools.partial(
        pl.kernel,
        out_type=out_type,
        mesh=_MESH,
        compiler_params=pltpu.CompilerParams(needs_layout_passes=False,
                                             use_tc_tiling_on_sc=False),
        scratch_types=[
            pltpu.VMEM((MAXCH, KCH), I32),
            pltpu.VMEM((MAXCH, KCH), I32),
            pltpu.VMEM((KCH, 64), F32),
            pltpu.VMEM((KCH, 64), F32),
            pltpu.VMEM((KCH, 64), F32),
            pltpu.VMEM((KCH, 64), F32),
            pltpu.VMEM((KCH, 64), F32),
            pltpu.VMEM((KCH, 16), F32),
            pltpu.VMEM((KCH, 64), F32),
            pltpu.VMEM((KCH, 16), F32),
            pltpu.VMEM((4, 16), F32),
            pltpu.VMEM_SHARED((ACC_BIG, 64), F32),
            pltpu.VMEM_SHARED((ACC_BIG, 16), F32),
            pltpu.SemaphoreType.DMA,
            pltpu.SemaphoreType.DMA,
            pltpu.SemaphoreType.DMA,
            pltpu.SemaphoreType.DMA,
            pltpu.SemaphoreType.DMA,
            pltpu.SemaphoreType.DMA,
            pltpu.SemaphoreType.DMA,
            pltpu.SemaphoreType.DMA,
        ],
    )
    def kern(*refs):
        ins = refs[:5 * njobs]
        outs = refs[5 * njobs:5 * njobs + 2 * njobs]
        (sidx, didx, hsb0, hdb0, hsb1, hdb1, msgb0, denb0, msgb1, denb1,
         attv, accn, accd, sem0, sem1, sem2, sem3,
         sem4, sem5, sem6, sem7) = refs[5 * njobs + 2 * njobs:]
        c = lax.axis_index("c")
        s = lax.axis_index("s")
        iot = lax.iota(I32, 16)
        zv = jnp.zeros((16,), F32)
        i15 = jnp.full((16,), 15, I32)

        flat = 0
        for core_id, jobs in enumerate(cfg):
            job_in = ins[5 * flat:]
            job_out = outs[2 * flat:]
            flat += len(jobs)

            @pl.when(c == core_id)
            def _(jobs=jobs, job_in=job_in, job_out=job_out):
                for ji, (nchunks, nd_acc) in enumerate(jobs):
                    hs_h, hd_h, src_h, dst_h, att_h = job_in[5 * ji:5 * ji + 5]
                    num_o, den_o = job_out[2 * ji:2 * ji + 2]
                    rpt = nd_acc // 16
                    _zero_rows(msgb0, KCH, 64)
                    _zero_rows(denb0, KCH, 16)
                    nfull, rem = rpt // KCH, rpt % KCH
                    for q in range(nfull):
                        pltpu.sync_copy(
                            msgb0, accn.at[pl.ds(s * rpt + q * KCH, KCH)])
                        pltpu.sync_copy(
                            denb0, accd.at[pl.ds(s * rpt + q * KCH, KCH)])
                    if rem:
                        pltpu.sync_copy(
                            msgb0.at[pl.ds(0, rem)],
                            accn.at[pl.ds(s * rpt + nfull * KCH, rem)])
                        pltpu.sync_copy(
                            denb0.at[pl.ds(0, rem)],
                            accd.at[pl.ds(s * rpt + nfull * KCH, rem)])
                    plsc.subcore_barrier()
                    chpad = -(-nchunks // 8) * 8
                    pltpu.sync_copy(src_h.at[pl.ds(s * chpad, chpad)],
                                    sidx.at[pl.ds(0, chpad)])
                    pltpu.sync_copy(dst_h.at[pl.ds(s * chpad, chpad)],
                                    didx.at[pl.ds(0, chpad)])
                    pltpu.sync_copy(att_h, attv)
                    att = [attv[i, :] for i in range(4)]

                    def compute_scatter(hsb, hdb, msgb, denb, msem, dsem,
                                        j, jj, first):
                        @pl.when(jj > first)
                        def _():
                            pltpu.make_async_copy(
                                msgb, accn.at[didx.at[j]], msem).wait()
                            pltpu.make_async_copy(
                                denb, accd.at[didx.at[j]], dsem).wait()

                        def one_edge(e):
                            hr = [hsb[e, pl.ds(16 * i, 16)] for i in range(4)]
                            dr = [hdb[e, pl.ds(16 * i, 16)] for i in range(4)]
                            sh = []
                            for h in range(2):
                                t0 = hr[2 * h] + dr[2 * h]
                                t1 = hr[2 * h + 1] + dr[2 * h + 1]
                                m0 = jnp.maximum(t0, 0.2 * t0)
                                m1 = jnp.maximum(t1, 0.2 * t1)
                                sh.append(m0 * att[2 * h] + m1 * att[2 * h + 1])
                            ex0 = jnp.exp(plsc.cumsum(sh[0])[i15])
                            ex1 = jnp.exp(plsc.cumsum(sh[1])[i15])
                            denb[e, :] = jnp.where(
                                iot == 0, ex0, jnp.where(iot == 1, ex1, zv))
                            msgb[e, pl.ds(0, 16)] = hr[0] * ex0
                            msgb[e, pl.ds(16, 16)] = hr[1] * ex0
                            msgb[e, pl.ds(32, 16)] = hr[2] * ex1
                            msgb[e, pl.ds(48, 16)] = hr[3] * ex1

                        def edge4(k, _):
                            for u in range(4):
                                one_edge(4 * k + u)
                            return 0

                        lax.fori_loop(0, KCH // 4, edge4, 0)
                        pltpu.async_copy(msgb, accn.at[didx.at[j]], msem,
                                         add=True)
                        pltpu.async_copy(denb, accd.at[didx.at[j]], dsem,
                                         add=True)

                    pltpu.async_copy(hs_h.at[sidx.at[0]], hsb0, sem0)
                    pltpu.async_copy(hd_h.at[didx.at[0]], hdb0, sem1)

                    def chunk2(j, _):
                        c0 = 2 * j
                        c1 = 2 * j + 1
                        pltpu.make_async_copy(
                            hs_h.at[sidx.at[c0]], hsb0, sem0).wait()
                        pltpu.make_async_copy(
                            hd_h.at[didx.at[c0]], hdb0, sem1).wait()
                        pltpu.async_copy(hs_h.at[sidx.at[c1]], hsb1, sem2)
                        pltpu.async_copy(hd_h.at[didx.at[c1]], hdb1, sem3)
                        compute_scatter(hsb0, hdb0, msgb0, denb0, sem4, sem5,
                                        c0, j, 0)
                        pltpu.make_async_copy(
                            hs_h.at[sidx.at[c1]], hsb1, sem2).wait()
                        pltpu.make_async_copy(
                            hd_h.at[didx.at[c1]], hdb1, sem3).wait()

                        @pl.when(c0 + 2 < nchunks)
                        def _():
                            pltpu.async_copy(
                                hs_h.at[sidx.at[c0 + 2]], hsb0, sem0)
                            pltpu.async_copy(
                                hd_h.at[didx.at[c0 + 2]], hdb0, sem1)

                        compute_scatter(hsb1, hdb1, msgb1, denb1, sem6, sem7,
                                        c1, j, 0)
                        return 0

                    lax.fori_loop(0, nchunks // 2, chunk2, 0)
                    pltpu.make_async_copy(
                        msgb0, accn.at[didx.at[nchunks - 2]], sem4).wait()
                    pltpu.make_async_copy(
                        denb0, accd.at[didx.at[nchunks - 2]], sem5).wait()
                    pltpu.make_async_copy(
                        msgb1, accn.at[didx.at[nchunks - 1]], sem6).wait()
                    pltpu.make_async_copy(
                        denb1, accd.at[didx.at[nchunks - 1]], sem7).wait()
                    plsc.subcore_barrier()
                    pltpu.sync_copy(accn.at[pl.ds(s * rpt, rpt)],
                                    num_o.at[pl.ds(s * rpt, rpt)])
                    pltpu.sync_copy(accd.at[pl.ds(s * rpt, rpt)],
                                    den_o.at[pl.ds(s * rpt, rpt)])
                    plsc.subcore_barrier()

    return kern


def _pool_kernel():
    @functools.partial(
        pl.kernel,
        out_type=[jax.ShapeDtypeStruct((768, 128), F32),
                  jax.ShapeDtypeStruct((768, 128), F32)],
        mesh=_MESH,
        compiler_params=pltpu.CompilerParams(needs_layout_passes=False,
                                             use_tc_tiling_on_sc=False),
        scratch_types=[
            pltpu.VMEM((128, 128), F32),
            pltpu.VMEM((8, 128), I32),
            pltpu.VMEM_SHARED((768, 128), F32),
        ],
    )
    def kern(x_h, bidx_h, out0, out1, xbuf, bptr, acc):
        c = lax.axis_index("c")
        s = lax.axis_index("s")
        wid = c * 16 + s
        _zero_rows(xbuf, 128, 128)
        pltpu.sync_copy(xbuf.at[pl.ds(0, 48)], acc.at[pl.ds(s * 48, 48)])
        plsc.subcore_barrier()
        pltpu.sync_copy(bidx_h.at[pl.ds(wid * 8, 8)], bptr)

        def chunk(j, _):
            pltpu.sync_copy(x_h.at[pl.ds(wid * 1024 + j * 128, 128)], xbuf)
            pltpu.sync_copy(xbuf, acc.at[bptr.at[j]], add=True)
            return 0

        lax.fori_loop(0, 8, chunk, 0)
        plsc.subcore_barrier()

        @pl.when(c == 0)
        def _():
            pltpu.sync_copy(acc.at[pl.ds(s * 48, 48)],
                            out0.at[pl.ds(s * 48, 48)])

        @pl.when(c == 1)
        def _():
            pltpu.sync_copy(acc.at[pl.ds(s * 48, 48)],
                            out1.at[pl.ds(s * 48, 48)])

    return kern


def _pad_rows(x, n):
    return jnp.concatenate(
        [x, jnp.zeros((n - x.shape[0],) + x.shape[1:], x.dtype)], axis=0)


def _edge_2d(ei, garb):
    e = ei.shape[1]
    t = e // 16
    nchunks = -(-t // KCH)
    nchunks += nchunks % 2
    chpad = -(-nchunks // 8) * 8
    tp = nchunks * KCH
    pad = 16 * tp - e
    src = jnp.concatenate([ei[0], jnp.zeros((pad,), I32)])
    dst = jnp.concatenate([ei[1], jnp.full((pad,), garb, I32)])
    rows = ((0, 0), (0, chpad - nchunks), (0, 0))
    src = jnp.pad(src.reshape(16, nchunks, KCH), rows).reshape(16 * chpad, KCH)
    dst = jnp.pad(dst.reshape(16, nchunks, KCH), rows,
                  constant_values=garb).reshape(16 * chpad, KCH)
    return src, dst, nchunks


def kernel(x_text, x_audio, x_video, x_z, lin_W, lin_b, conv_Wl, conv_Wr,
           conv_att, conv_bias, proj_W0, proj_b0, proj_W1, proj_b1,
           ei_ta, ei_at, ei_tv, ei_vt, ei_av, ei_va, ei_tz, ei_zt, ei_az,
           ei_za, ei_vz, ei_zv, ei_zz, batch_text, batch_audio, batch_video):
    eis = [ei_ta, ei_at, ei_tv, ei_vt, ei_av, ei_va, ei_tz, ei_zt, ei_az,
           ei_za, ei_vz, ei_zv, ei_zz]

    xstack = jnp.stack([_pad_rows(x_text, NPAD), _pad_rows(x_audio, NPAD),
                        _pad_rows(x_video, NPAD), _pad_rows(x_z, NPAD)])
    segs_p = [jnp.concatenate([b, jnp.full((NPAD - NBIG,), G, I32)])
              for b in (batch_text, batch_audio, batch_video)]
    segf = jnp.stack(segs_p).astype(F32).reshape(3, 20, 512)
    segrow = segf.reshape(60, 1, 512)
    segcol = segf.reshape(60, 512, 1)
    edges = []
    for i, ei in enumerate(eis):
        garb = GARB_Z if _DST_IDS[i] == 3 else GARB_BIG
        edges.append(_edge_2d(ei, garb))

    post = _positions(segrow, segcol)
    lw = jnp.stack([lin_W[0], lin_W[1], lin_W[2], lin_W[0]])
    lb = jnp.broadcast_to(
        jnp.stack([lin_b[0], lin_b[1], lin_b[2], lin_b[0]])[:, None, :],
        (4, 8, 128))
    flags = jnp.asarray([1, 1, 1, 0], I32)
    h = _lin_pe(xstack, lw, lb, post, flags)

    stab = jnp.asarray(_SRC_IDS + _DST_IDS, I32)
    launches = [
        (("at",), ("ta",)),
        (("vt",), ("tv",)),
        (("va",), ("av",)),
        (("zt", "tz"), ("za", "az")),
        (("zv", "vz"), ("zz",)),
    ]
    dst_groups = {"t": ["at", "vt", "zt"], "a": ["ta", "va", "za"],
                  "v": ["tv", "av", "zv"], "z": ["tz", "az", "vz", "zz"]}

    for l in range(2):
        w26 = jnp.concatenate([conv_Wl[l], conv_Wr[l]], axis=0)
        t26h = [_project26(stab, h, w26[:, :, :64]),
                _project26(stab, h, w26[:, :, 64:])]
        results = {}
        for cfg_jobs in launches:
            cfg = []
            args = []
            names = []
            for core_jobs in cfg_jobs:
                core_cfg = []
                for nm in core_jobs:
                    i = _ENAME2IDX[nm]
                    src2d, dst2d, nchunks = edges[i]
                    nd_acc = ACC_Z if _DST_IDS[i] == 3 else ACC_BIG
                    for half in range(2):
                        core_cfg.append((nchunks, nd_acc))
                        args += [t26h[half][i], t26h[half][13 + i],
                                 src2d, dst2d,
                                 conv_att[l, i, 2 * half:2 * half + 2]
                                 .reshape(4, 16)]
                        names.append((nm, half))
                cfg.append(tuple(core_cfg))
            outs = _edge_kernel(tuple(cfg))(*args)
            for k, key in enumerate(names):
                results[key] = (outs[2 * k], outs[2 * k + 1])

        news = {}
        for tname, group in dst_groups.items():
            pairs = []
            for nm in group:
                n0, d0 = results[(nm, 0)]
                n1, d1 = results[(nm, 1)]
                num = jnp.concatenate([n0, n1], axis=1)
                den = jnp.pad(
                    jnp.concatenate([d0[:, 0:2], d1[:, 0:2]], axis=1),
                    ((0, 0), (0, 12)))
                pairs.append((num, den))
            bsum = sum(conv_bias[l, _ENAME2IDX[nm]] for nm in group)
            bias8 = jnp.broadcast_to(bsum[None, :], (8, 128))
            nd_acc = ACC_Z if tname == "z" else ACC_BIG
            news[tname] = _combine(nd_acc, pairs, bias8, 1.0 / len(group))
        h = jnp.stack([news["t"], news["a"], news["v"],
                       _pad_rows(news["z"], NPAD)])

    xcat = jnp.concatenate(
        [h[0], h[1], h[2], jnp.zeros((2048, 128), F32)], axis=0)
    bidx = jnp.concatenate(
        segs_p + [jnp.full((2048,), G, I32)]).reshape(256, 128)
    pool0, pool1 = _pool_kernel()(xcat, bidx)
    b0 = jnp.broadcast_to(proj_b0[None, :], (8, 128))
    b1 = jnp.broadcast_to(proj_b1[None, :], (8, 128))
    out = _loss(pool0, pool1, segrow, proj_W0, b0, proj_W1, b1)
    return out[0, 0]
